# baseline (speedup 1.0000x reference)
import functools

import jax
import jax.numpy as jnp
from jax import lax
from jax.experimental import pallas as pl
from jax.experimental.pallas import tpu as pltpu
from jax.experimental.pallas import tpu_sc as plsc

ROWS = 16
COLS = 4096
LANES = 16
NCHUNKS = COLS // LANES

_mesh = plsc.VectorSubcoreMesh(
    core_axis_name="c", subcore_axis_name="s", num_cores=1
)


@functools.partial(
    pl.kernel,
    out_type=jax.ShapeDtypeStruct((ROWS, COLS), jnp.int32),
    mesh=_mesh,
    scratch_types=[
        pltpu.VMEM((COLS,), jnp.int32),
        pltpu.VMEM((COLS,), jnp.int32),
    ],
    compiler_params=pltpu.CompilerParams(needs_layout_passes=False),
)
def _cumsum_rows(x_hbm, out_hbm, x_v, o_v):
    wid = lax.axis_index("s")

    @pl.when(wid < ROWS)
    def _():
        pltpu.sync_copy(x_hbm.at[wid], x_v)
        last = jnp.full((LANES,), LANES - 1, jnp.int32)

        def body(i, carry):
            v = x_v[pl.ds(i * LANES, LANES)]
            s = plsc.cumsum(v)
            o_v[pl.ds(i * LANES, LANES)] = s + carry
            total = s.at[last].get(mode="promise_in_bounds")
            return carry + total

        lax.fori_loop(0, NCHUNKS, body, jnp.full((LANES,), -1, jnp.int32), unroll=8)
        pltpu.sync_copy(o_v, out_hbm.at[wid])


def kernel(pad_masks):
    return _cumsum_rows(pad_masks.astype(jnp.int32))


# speedup vs baseline: 1.0460x; 1.0460x over previous
import functools

import jax
import jax.numpy as jnp
from jax import lax
from jax.experimental import pallas as pl
from jax.experimental.pallas import tpu as pltpu
from jax.experimental.pallas import tpu_sc as plsc

ROWS = 16
COLS = 4096
LANES = 16
NCHUNKS = COLS // LANES

_mesh = plsc.VectorSubcoreMesh(
    core_axis_name="c", subcore_axis_name="s", num_cores=1
)


@functools.partial(
    pl.kernel,
    out_type=jax.ShapeDtypeStruct((ROWS, COLS), jnp.int32),
    mesh=_mesh,
    scratch_types=[
        pltpu.VMEM((COLS,), jnp.int32),
        pltpu.VMEM((COLS,), jnp.int32),
    ],
    compiler_params=pltpu.CompilerParams(needs_layout_passes=False),
)
def _cumsum_rows(x_hbm, out_hbm, x_v, o_v):
    wid = lax.axis_index("s")

    @pl.when(wid < ROWS)
    def _():
        pltpu.sync_copy(x_hbm.at[wid], x_v)
        last = jnp.full((LANES,), LANES - 1, jnp.int32)

        def body(i, carry):
            v = x_v[pl.ds(i * LANES, LANES)]
            s = plsc.cumsum(v)
            o_v[pl.ds(i * LANES, LANES)] = s + carry
            total = s.at[last].get(mode="promise_in_bounds")
            return carry + total

        lax.fori_loop(0, NCHUNKS, body, jnp.full((LANES,), -1, jnp.int32))
        pltpu.sync_copy(o_v, out_hbm.at[wid])


def kernel(pad_masks):
    return _cumsum_rows(pad_masks.astype(jnp.int32))


# speedup vs baseline: 1.0775x; 1.0301x over previous
import functools

import jax
import jax.numpy as jnp
from jax import lax
from jax.experimental import pallas as pl
from jax.experimental.pallas import tpu as pltpu
from jax.experimental.pallas import tpu_sc as plsc

ROWS = 16
COLS = 4096
LANES = 16
BYTES_PER_CHUNK = 4 * LANES
NCHUNKS = COLS // BYTES_PER_CHUNK

_mesh = plsc.VectorSubcoreMesh(
    core_axis_name="c", subcore_axis_name="s", num_cores=1
)


@functools.partial(
    pl.kernel,
    out_type=jax.ShapeDtypeStruct((ROWS, COLS), jnp.int32),
    mesh=_mesh,
    scratch_types=[
        pltpu.VMEM((COLS,), jnp.int8),
        pltpu.VMEM((COLS,), jnp.int32),
    ],
    compiler_params=pltpu.CompilerParams(needs_layout_passes=False),
)
def _cumsum_rows(x_hbm, out_hbm, x_v, o_v):
    wid = lax.axis_index("s")

    @pl.when(wid < ROWS)
    def _():
        pltpu.sync_copy(x_hbm.at[pl.ds(wid * COLS, COLS)], x_v)
        lane = lax.iota(jnp.int32, LANES)
        idx0 = lane * 4
        last = jnp.full((LANES,), LANES - 1, jnp.int32)

        def body(i, carry):
            chunk = x_v[pl.ds(i * BYTES_PER_CHUNK, BYTES_PER_CHUNK)]
            w = plsc.bitcast(chunk, jnp.int32)
            p = w * jnp.int32(0x01010101)
            t = lax.shift_right_logical(p, jnp.int32(24))
            ws = plsc.cumsum(t)
            base = carry + (ws - t)
            idx = idx0 + i * BYTES_PER_CHUNK
            for k in range(4):
                if k < 3:
                    val = lax.shift_right_logical(p, jnp.int32(8 * k))
                    val = val & jnp.int32(0xFF)
                else:
                    val = t
                plsc.store_scatter(o_v, [idx + k], val + base)
            total = ws.at[last].get(mode="promise_in_bounds")
            return carry + total

        lax.fori_loop(
            0, NCHUNKS, body, jnp.full((LANES,), -1, jnp.int32)
        )
        pltpu.sync_copy(o_v, out_hbm.at[wid])


def kernel(pad_masks):
    return _cumsum_rows(pad_masks.view(jnp.int8).reshape(ROWS * COLS))
